# Initial kernel scaffold; baseline (speedup 1.0000x reference)
import jax
import jax.numpy as jnp
from jax.experimental import pallas as pl


def kernel(x, edge_list, W, b):
    raise NotImplementedError("write your pallas kernel here")


# speedup vs baseline: 3.1209x; 3.1209x over previous
import functools

import jax
import jax.numpy as jnp
from jax import lax
from jax.experimental import pallas as pl
from jax.experimental.pallas import tpu as pltpu
from jax.experimental.pallas import tpu_sc as plsc

NN = 10000
EE = 160000
DD = 128

NCORES = 2
NSUB = 16
NTILES = NCORES * NSUB
EPAD = 163840
EDGES_PER_TILE = EPAD // NTILES
CHUNK = 128
NCHUNKS = EDGES_PER_TILE // CHUNK
ACC_ROWS = 10240
ZROWS = 640


def _sc_body(keys_hbm, kprev_hbm, x_hbm, zeros_hbm, p_hbm,
             keys_v, kprev_v, tidx_v, sidx_v, rows_v, outbuf_v, accum_sh,
             gsem):
    c = lax.axis_index("c")
    s = lax.axis_index("s")
    lane = lax.iota(jnp.int32, 16)

    pltpu.sync_copy(zeros_hbm, accum_sh.at[pl.ds(s * ZROWS, ZROWS)])
    plsc.subcore_barrier()

    tile_base = (c * NSUB + s) * EDGES_PER_TILE
    trash = 10000 + s

    def chunk_body(k, carry):
        base = tile_base + k * CHUNK
        pltpu.sync_copy(keys_hbm.at[pl.ds(base, CHUNK)], keys_v)
        pltpu.sync_copy(kprev_hbm.at[pl.ds(base, CHUNK)], kprev_v)

        def grp(j, carry2):
            kv = keys_v[pl.ds(j * 16, 16)]
            kp = kprev_v[pl.ds(j * 16, 16)]
            keep = kv != kp
            t = lax.shift_right_logical(kv, 14)
            src = lax.bitwise_and(kv, 16383)
            dummy_t = lax.bitwise_and(base + j * 16 + lane, 8191)
            tidx_v[pl.ds(j * 16, 16)] = jnp.where(keep, t, dummy_t)
            sidx_v[pl.ds(j * 16, 16)] = jnp.where(keep, src, trash)
            return carry2

        lax.fori_loop(0, CHUNK // 16, grp, 0, unroll=True)

        pltpu.async_copy(x_hbm.at[tidx_v], rows_v, gsem).wait()
        pltpu.sync_copy(rows_v, accum_sh.at[sidx_v], add=True)
        return carry

    lax.fori_loop(0, NCHUNKS, chunk_body, 0)
    plsc.subcore_barrier()

    row0 = s * 624
    for q in range(5):
        r = row0 + q * 128
        pltpu.sync_copy(accum_sh.at[pl.ds(r, 128)], outbuf_v)
        pltpu.sync_copy(outbuf_v, p_hbm.at[c, pl.ds(r, 128)])


@jax.jit
def _sc_segment_sum(keys_p, kprev_p, x, zeros_rows):
    mesh = plsc.VectorSubcoreMesh(core_axis_name="c", subcore_axis_name="s")
    kern = pl.kernel(
        _sc_body,
        out_type=jax.ShapeDtypeStruct((NCORES, NN, DD), jnp.float32),
        mesh=mesh,
        scratch_types=[
            pltpu.VMEM((CHUNK,), jnp.int32),
            pltpu.VMEM((CHUNK,), jnp.int32),
            pltpu.VMEM((CHUNK,), jnp.int32),
            pltpu.VMEM((CHUNK,), jnp.int32),
            pltpu.VMEM((CHUNK, DD), jnp.float32),
            pltpu.VMEM((CHUNK, DD), jnp.float32),
            pltpu.VMEM_SHARED((ACC_ROWS, DD), jnp.float32),
            pltpu.SemaphoreType.DMA,
        ],
    )
    return kern(keys_p, kprev_p, x, zeros_rows)


def _mm_body(p_ref, w_ref, b_ref, o_ref):
    xs = p_ref[0] + p_ref[1]
    o_ref[...] = jnp.dot(xs, w_ref[...],
                         preferred_element_type=jnp.float32) + b_ref[...]


@jax.jit
def _combine_matmul(p, w, b2):
    bm = 1000
    return pl.pallas_call(
        _mm_body,
        grid=(NN // bm,),
        in_specs=[
            pl.BlockSpec((NCORES, bm, DD), lambda i: (0, i, 0)),
            pl.BlockSpec((DD, DD), lambda i: (0, 0)),
            pl.BlockSpec((1, DD), lambda i: (0, 0)),
        ],
        out_specs=pl.BlockSpec((bm, DD), lambda i: (i, 0)),
        out_shape=jax.ShapeDtypeStruct((NN, DD), jnp.float32),
    )(p, w, b2)


def kernel(x, edge_list, W, b):
    src = edge_list[0].astype(jnp.int32)
    dst = edge_list[1].astype(jnp.int32)
    keys = dst * 16384 + src
    ks = jnp.sort(keys)
    kprev = jnp.concatenate([jnp.full((1,), -1, jnp.int32), ks[:-1]])
    zpad = jnp.zeros((EPAD - EE,), jnp.int32)
    ksp = jnp.concatenate([ks, zpad])
    kpp = jnp.concatenate([kprev, zpad])
    zeros_rows = jnp.zeros((ZROWS, DD), jnp.float32)
    partials = _sc_segment_sum(ksp, kpp, x, zeros_rows)
    return _combine_matmul(partials, W, b.reshape(1, DD))


# speedup vs baseline: 4.0010x; 1.2820x over previous
import functools

import jax
import jax.numpy as jnp
from jax import lax
from jax.experimental import pallas as pl
from jax.experimental.pallas import tpu as pltpu
from jax.experimental.pallas import tpu_sc as plsc

NN = 10000
EE = 160000
DD = 128

NCORES = 2
NSUB = 16
NTILES = NCORES * NSUB
EPAD = 163840
EDGES_PER_TILE = EPAD // NTILES
CHUNK = 128
NCHUNKS = EDGES_PER_TILE // CHUNK
ACC_ROWS = 10240
ZROWS = 640


RING = 2


def _sc_body(keys_hbm, kprev_hbm, x_hbm, zeros_hbm, p_hbm,
             keys_v, kprev_v, tidx_v, sidx_v, rows_v, accum_sh,
             gsem0, gsem1):
    c = lax.axis_index("c")
    s = lax.axis_index("s")
    lane = lax.iota(jnp.int32, 16)
    gsems = [gsem0, gsem1]

    pltpu.sync_copy(zeros_hbm, accum_sh.at[pl.ds(s * ZROWS, ZROWS)])

    tile_base = (c * NSUB + s) * EDGES_PER_TILE
    trash = 10000 + s

    tile_row0 = (c * NSUB + s) * NCHUNKS
    pltpu.sync_copy(keys_hbm.at[pl.ds(tile_row0, NCHUNKS)], keys_v)
    pltpu.sync_copy(kprev_hbm.at[pl.ds(tile_row0, NCHUNKS)], kprev_v)
    plsc.subcore_barrier()

    def compute_indices(row, b):
        def grp(j, carry2):
            kv = keys_v[row, pl.ds(j * 16, 16)]
            kp = kprev_v[row, pl.ds(j * 16, 16)]
            keep = kv != kp
            t = lax.shift_right_logical(kv, 14)
            src = lax.bitwise_and(kv, 16383)
            dummy_t = lax.bitwise_and(tile_base + row * 128 + j * 16 + lane,
                                      8191)
            tidx_v[b, pl.ds(j * 16, 16)] = jnp.where(keep, t, dummy_t)
            sidx_v[b, pl.ds(j * 16, 16)] = jnp.where(keep, src, trash)
            return carry2
        lax.fori_loop(0, CHUNK // 16, grp, 0)

    handles = [None] * NCHUNKS
    for k in range(NCHUNKS):
        b = k % RING
        if k >= RING:
            handles[k - RING].wait()
            pltpu.sync_copy(rows_v.at[b], accum_sh.at[sidx_v.at[b]],
                            add=True)
        compute_indices(k, b)
        handles[k] = pltpu.async_copy(x_hbm.at[tidx_v.at[b]], rows_v.at[b],
                                      gsems[b])
    for k in range(NCHUNKS - RING, NCHUNKS):
        b = k % RING
        handles[k].wait()
        pltpu.sync_copy(rows_v.at[b], accum_sh.at[sidx_v.at[b]], add=True)
    plsc.subcore_barrier()

    row0 = s * 624
    for q in range(5):
        r = row0 + q * 128
        pltpu.sync_copy(accum_sh.at[pl.ds(r, 128)], rows_v.at[0])
        pltpu.sync_copy(rows_v.at[0], p_hbm.at[c, pl.ds(r, 128)])


@jax.jit
def _sc_segment_sum(keys_p, kprev_p, x, zeros_rows):
    mesh = plsc.VectorSubcoreMesh(core_axis_name="c", subcore_axis_name="s")
    kern = pl.kernel(
        _sc_body,
        out_type=jax.ShapeDtypeStruct((NCORES, NN, DD), jnp.float32),
        mesh=mesh,
        scratch_types=[
            pltpu.VMEM((NCHUNKS, CHUNK), jnp.int32),
            pltpu.VMEM((NCHUNKS, CHUNK), jnp.int32),
            pltpu.VMEM((RING, CHUNK), jnp.int32),
            pltpu.VMEM((RING, CHUNK), jnp.int32),
            pltpu.VMEM((RING, CHUNK, DD), jnp.float32),
            pltpu.VMEM_SHARED((ACC_ROWS, DD), jnp.float32),
            pltpu.SemaphoreType.DMA,
            pltpu.SemaphoreType.DMA,
        ],
    )
    return kern(keys_p, kprev_p, x, zeros_rows)


def _mm_body(p_ref, w_ref, b_ref, o_ref):
    xs = p_ref[0] + p_ref[1]
    o_ref[...] = jnp.dot(xs, w_ref[...],
                         preferred_element_type=jnp.float32) + b_ref[...]


@jax.jit
def _combine_matmul(p, w, b2):
    bm = 1000
    return pl.pallas_call(
        _mm_body,
        grid=(NN // bm,),
        in_specs=[
            pl.BlockSpec((NCORES, bm, DD), lambda i: (0, i, 0)),
            pl.BlockSpec((DD, DD), lambda i: (0, 0)),
            pl.BlockSpec((1, DD), lambda i: (0, 0)),
        ],
        out_specs=pl.BlockSpec((bm, DD), lambda i: (i, 0)),
        out_shape=jax.ShapeDtypeStruct((NN, DD), jnp.float32),
    )(p, w, b2)


def kernel(x, edge_list, W, b):
    src = edge_list[0].astype(jnp.int32)
    dst = edge_list[1].astype(jnp.int32)
    keys = dst * 16384 + src
    ks = jnp.sort(keys)
    kprev = jnp.concatenate([jnp.full((1,), -1, jnp.int32), ks[:-1]])
    zpad = jnp.zeros((EPAD - EE,), jnp.int32)
    ksp = jnp.concatenate([ks, zpad]).reshape(EPAD // CHUNK, CHUNK)
    kpp = jnp.concatenate([kprev, zpad]).reshape(EPAD // CHUNK, CHUNK)
    zeros_rows = jnp.zeros((ZROWS, DD), jnp.float32)
    partials = _sc_segment_sum(ksp, kpp, x, zeros_rows)
    return _combine_matmul(partials, W, b.reshape(1, DD))


# speedup vs baseline: 6.3525x; 1.5877x over previous
import functools

import jax
import jax.numpy as jnp
from jax import lax
from jax.experimental import pallas as pl
from jax.experimental.pallas import tpu as pltpu
from jax.experimental.pallas import tpu_sc as plsc

NN = 10000
EE = 160000
DD = 128

NCORES = 2
NSUB = 16
NTILES = NCORES * NSUB
EPAD = 163840
EDGES_PER_TILE = EPAD // NTILES
CHUNK = 128
NCHUNKS = EDGES_PER_TILE // CHUNK
ACC_ROWS = 10240
ZROWS = 640


RING = 2


def _sc_body(keys_hbm, kprev_hbm, x_hbm, zeros_hbm, p_hbm,
             keys_v, kprev_v, tidx_v, sidx_v, rows_v, accum_sh,
             gsem0, gsem1):
    c = lax.axis_index("c")
    s = lax.axis_index("s")
    lane = lax.iota(jnp.int32, 16)
    gsems = [gsem0, gsem1]

    pltpu.sync_copy(zeros_hbm, accum_sh.at[pl.ds(s * ZROWS, ZROWS)])

    tile_base = (c * NSUB + s) * EDGES_PER_TILE
    trash = 10000 + s

    tile_row0 = (c * NSUB + s) * NCHUNKS
    pltpu.sync_copy(keys_hbm.at[pl.ds(tile_row0, NCHUNKS)], keys_v)
    pltpu.sync_copy(kprev_hbm.at[pl.ds(tile_row0, NCHUNKS)], kprev_v)
    plsc.subcore_barrier()

    def compute_indices(row, b):
        def grp(j, carry2):
            kv = keys_v[row, pl.ds(j * 16, 16)]
            kp = kprev_v[row, pl.ds(j * 16, 16)]
            keep = kv != kp
            t = lax.shift_right_logical(kv, 14)
            src = lax.bitwise_and(kv, 16383)
            dummy_t = lax.bitwise_and(tile_base + row * 128 + j * 16 + lane,
                                      8191)
            tidx_v[b, pl.ds(j * 16, 16)] = jnp.where(keep, t, dummy_t)
            sidx_v[b, pl.ds(j * 16, 16)] = jnp.where(keep, src, trash)
            return carry2
        lax.fori_loop(0, CHUNK // 16, grp, 0)

    handles = [None] * NCHUNKS
    for k in range(NCHUNKS):
        b = k % RING
        if k >= RING:
            handles[k - RING].wait()
            pltpu.sync_copy(rows_v.at[b], accum_sh.at[sidx_v.at[b]],
                            add=True)
        compute_indices(k, b)
        handles[k] = pltpu.async_copy(x_hbm.at[tidx_v.at[b]], rows_v.at[b],
                                      gsems[b])
    for k in range(NCHUNKS - RING, NCHUNKS):
        b = k % RING
        handles[k].wait()
        pltpu.sync_copy(rows_v.at[b], accum_sh.at[sidx_v.at[b]], add=True)
    plsc.subcore_barrier()

    row0 = s * 624
    for q in range(5):
        r = row0 + q * 128
        pltpu.sync_copy(accum_sh.at[pl.ds(r, 128)], rows_v.at[0])
        pltpu.sync_copy(rows_v.at[0], p_hbm.at[c, pl.ds(r, 128)])


@jax.jit
def _sc_segment_sum(keys_p, kprev_p, x, zeros_rows):
    mesh = plsc.VectorSubcoreMesh(core_axis_name="c", subcore_axis_name="s")
    kern = pl.kernel(
        _sc_body,
        out_type=jax.ShapeDtypeStruct((NCORES, NN, DD), jnp.float32),
        mesh=mesh,
        scratch_types=[
            pltpu.VMEM((NCHUNKS, CHUNK), jnp.int32),
            pltpu.VMEM((NCHUNKS, CHUNK), jnp.int32),
            pltpu.VMEM((RING, CHUNK), jnp.int32),
            pltpu.VMEM((RING, CHUNK), jnp.int32),
            pltpu.VMEM((RING, CHUNK, DD), jnp.float32),
            pltpu.VMEM_SHARED((ACC_ROWS, DD), jnp.float32),
            pltpu.SemaphoreType.DMA,
            pltpu.SemaphoreType.DMA,
        ],
    )
    return kern(keys_p, kprev_p, x, zeros_rows)


def _mm_body(p_ref, w_ref, b_ref, o_ref):
    xs = p_ref[0] + p_ref[1]
    o_ref[...] = jnp.dot(xs, w_ref[...],
                         preferred_element_type=jnp.float32) + b_ref[...]


@jax.jit
def _combine_matmul(p, w, b2):
    bm = 1000
    return pl.pallas_call(
        _mm_body,
        grid=(NN // bm,),
        in_specs=[
            pl.BlockSpec((NCORES, bm, DD), lambda i: (0, i, 0)),
            pl.BlockSpec((DD, DD), lambda i: (0, 0)),
            pl.BlockSpec((1, DD), lambda i: (0, 0)),
        ],
        out_specs=pl.BlockSpec((bm, DD), lambda i: (i, 0)),
        out_shape=jax.ShapeDtypeStruct((NN, DD), jnp.float32),
    )(p, w, b2)


def kernel(x, edge_list, W, b):
    src = edge_list[0].astype(jnp.int32)
    dst = edge_list[1].astype(jnp.int32)
    keys = dst * 16384 + src
    ks = lax.sort(keys, is_stable=False)
    kprev = jnp.concatenate([jnp.full((1,), -1, jnp.int32), ks[:-1]])
    zpad = jnp.zeros((EPAD - EE,), jnp.int32)
    ksp = jnp.concatenate([ks, zpad]).reshape(EPAD // CHUNK, CHUNK)
    kpp = jnp.concatenate([kprev, zpad]).reshape(EPAD // CHUNK, CHUNK)
    zeros_rows = jnp.zeros((ZROWS, DD), jnp.float32)
    partials = _sc_segment_sum(ksp, kpp, x, zeros_rows)
    return _combine_matmul(partials, W, b.reshape(1, DD))
